# Initial kernel scaffold; baseline (speedup 1.0000x reference)
import jax
import jax.numpy as jnp
from jax.experimental import pallas as pl


def kernel(x, edge_index, edge_attr, batch, nn1_w1, nn1_b1, nn1_w2, nn1_b2, root1, bias1, nn2_w1, nn2_b1, nn2_w2, nn2_b2, root2, bias2, nn3_w1, nn3_b1, nn3_w2, nn3_b2, root3, bias3, fc1_w, fc1_b, fc2_w, fc2_b, fc3_w, fc3_b):
    raise NotImplementedError("write your pallas kernel here")


# speedup vs baseline: 1.1154x; 1.1154x over previous
import functools

import jax
import jax.numpy as jnp
from jax import lax
from jax.experimental import pallas as pl
from jax.experimental.pallas import tpu as pltpu
from jax.experimental.pallas import tpu_sc as plsc

N = 10000
E = 40000
G = 256

NC, NS = 2, 16
NW = NC * NS
EPW = 1280
E_PAD = NW * EPW
CHUNK = 128
NCH = EPW // CHUNK
N_PAD = 10240
ROWS_PT = N_PAD // NS
TE = 256
TN = 512
NGRID_N = N_PAD // TN

def _sc_mesh():
  return plsc.VectorSubcoreMesh(core_axis_name="c", subcore_axis_name="s",
                                num_cores=NC, num_subcores=NS)


def _sc_gather(x_pad, src2d, d):
  @functools.partial(
      pl.kernel,
      out_type=jax.ShapeDtypeStruct((E_PAD, d), jnp.float32),
      mesh=_sc_mesh(),
      compiler_params=pltpu.CompilerParams(use_tc_tiling_on_sc=False),
      scratch_types=[
          pltpu.VMEM((NCH, CHUNK), jnp.int32),
          pltpu.VMEM((EPW, d), jnp.float32),
          pltpu.SemaphoreType.DMA,
      ],
  )
  def k(x_hbm, src_hbm, out_hbm, idx_v, rows_v, sem):
    wid = lax.axis_index("c") * NS + lax.axis_index("s")
    pltpu.sync_copy(src_hbm.at[wid], idx_v)
    descs = [
        pltpu.async_copy(x_hbm.at[idx_v.at[j]],
                         rows_v.at[pl.ds(j * CHUNK, CHUNK)], sem)
        for j in range(NCH)
    ]
    for de in descs:
      de.wait()
    pltpu.sync_copy(rows_v, out_hbm.at[pl.ds(wid * EPW, EPW)])

  return k(x_pad, src2d)


def _sc_scatter(msg, dst2d, zeros_nd, d):
  @functools.partial(
      pl.kernel,
      out_type=jax.ShapeDtypeStruct((2 * N_PAD, d), jnp.float32),
      mesh=_sc_mesh(),
      compiler_params=pltpu.CompilerParams(use_tc_tiling_on_sc=False),
      scratch_types=[
          pltpu.VMEM((NCH, CHUNK), jnp.int32),
          pltpu.VMEM((EPW, d), jnp.float32),
          pltpu.VMEM_SHARED((N_PAD, d), jnp.float32),
          pltpu.SemaphoreType.DMA,
      ],
  )
  def k(msg_hbm, dst_hbm, z_hbm, out_hbm, idx_v, msg_v, agg_sh, sem):
    cid = lax.axis_index("c")
    sid = lax.axis_index("s")
    wid = cid * NS + sid
    pltpu.sync_copy(z_hbm.at[pl.ds(sid * ROWS_PT, ROWS_PT)],
                    agg_sh.at[pl.ds(sid * ROWS_PT, ROWS_PT)])
    plsc.subcore_barrier()
    pltpu.sync_copy(dst_hbm.at[wid], idx_v)
    pltpu.sync_copy(msg_hbm.at[pl.ds(wid * EPW, EPW)], msg_v)
    for j in range(NCH):
      pltpu.sync_copy(msg_v.at[pl.ds(j * CHUNK, CHUNK)],
                      agg_sh.at[idx_v.at[j]], add=True)
    plsc.subcore_barrier()
    pltpu.sync_copy(agg_sh.at[pl.ds(sid * ROWS_PT, ROWS_PT)],
                    out_hbm.at[pl.ds(cid * N_PAD + sid * ROWS_PT, ROWS_PT)])

  return k(msg, dst2d, zeros_nd)


def _tc_message(eattr8, xsrc, w1p, b1r, w2p, b2p, dp, mo):
  grid = E_PAD // TE

  def body(ea_ref, xs_ref, w1_ref, b1_ref, w2_ref, b2_ref, out_ref):
    i = pl.program_id(0)
    bf = jnp.bfloat16
    h = jnp.maximum(
        jnp.dot(ea_ref[...].astype(bf), w1_ref[...].astype(bf),
                preferred_element_type=jnp.float32) + b1_ref[...], 0.0)
    ew = jnp.dot(h.astype(bf), w2_ref[...].astype(bf),
                 preferred_element_type=jnp.float32) + b2_ref[...]
    ew3 = ew.reshape(TE, dp, mo)
    xs3 = xs_ref[...].reshape(TE, dp, 1)
    msg = jnp.sum(ew3 * xs3, axis=1)
    rows = i * TE + lax.broadcasted_iota(jnp.int32, (TE, 1), 0)
    out_ref[...] = jnp.where(rows < E, msg, 0.0)

  return pl.pallas_call(
      body,
      grid=(grid,),
      in_specs=[
          pl.BlockSpec((TE, 8), lambda i: (i, 0)),
          pl.BlockSpec((TE, dp), lambda i: (i, 0)),
          pl.BlockSpec((8, 128), lambda i: (0, 0)),
          pl.BlockSpec((1, 128), lambda i: (0, 0)),
          pl.BlockSpec((128, dp * mo), lambda i: (0, 0)),
          pl.BlockSpec((1, dp * mo), lambda i: (0, 0)),
      ],
      out_specs=pl.BlockSpec((TE, mo), lambda i: (i, 0)),
      out_shape=jax.ShapeDtypeStruct((E_PAD, mo), jnp.float32),
  )(eattr8, xsrc, w1p, b1r, w2p, b2p)


def _tc_update(x_pad, agg, root_p, bias_r, dp_in, mo):
  def body(x_ref, a0_ref, a1_ref, r_ref, b_ref, out_ref):
    i = pl.program_id(0)
    v = (jnp.dot(x_ref[...].astype(jnp.bfloat16), r_ref[...].astype(jnp.bfloat16),
                 preferred_element_type=jnp.float32)
         + b_ref[...] + a0_ref[...] + a1_ref[...])
    act = jnp.where(v > 0, v, jnp.exp(jnp.minimum(v, 0.0)) - 1.0)
    rows = i * TN + lax.broadcasted_iota(jnp.int32, (TN, 1), 0)
    out_ref[...] = jnp.where(rows < N, act, 0.0)

  return pl.pallas_call(
      body,
      grid=(NGRID_N,),
      in_specs=[
          pl.BlockSpec((TN, dp_in), lambda i: (i, 0)),
          pl.BlockSpec((TN, mo), lambda i: (i, 0)),
          pl.BlockSpec((TN, mo), lambda i: (i + NGRID_N, 0)),
          pl.BlockSpec((dp_in, mo), lambda i: (0, 0)),
          pl.BlockSpec((1, mo), lambda i: (0, 0)),
      ],
      out_specs=pl.BlockSpec((TN, mo), lambda i: (i, 0)),
      out_shape=jax.ShapeDtypeStruct((N_PAD, mo), jnp.float32),
  )(x_pad, agg, agg, root_p, bias_r)


def _tc_pool_head(x3, batch2d, fc1_w, fc1_b, fc2_w, fc2_b, fc3_w, fc3_b):
  def body(x_ref, b_ref, w1_ref, b1_ref, w2_ref, b2_ref, w3_ref, b3_ref,
           out_ref, sums, cnt):
    i = pl.program_id(0)

    @pl.when(i == 0)
    def _():
      sums[...] = jnp.zeros_like(sums)
      cnt[...] = jnp.zeros_like(cnt)

    oh = (b_ref[...] == lax.broadcasted_iota(jnp.int32, (TN, G), 1)
          ).astype(jnp.float32)
    sums[...] += lax.dot_general(oh, x_ref[...], (((0,), (0,)), ((), ())),
                                 precision=lax.Precision.HIGHEST,
                                 preferred_element_type=jnp.float32)
    cnt[...] += lax.dot_general(oh, jnp.ones((TN, 1), jnp.float32),
                                (((0,), (0,)), ((), ())),
                                precision=lax.Precision.HIGHEST,
                                preferred_element_type=jnp.float32)

    @pl.when(i == NGRID_N - 1)
    def _():
      bf = jnp.bfloat16
      mean = sums[...] / jnp.maximum(cnt[...], 1.0)
      h1 = jnp.dot(mean.astype(bf), w1_ref[...].astype(bf),
                   preferred_element_type=jnp.float32) + b1_ref[...]
      h1 = jnp.where(h1 > 0, h1, jnp.exp(jnp.minimum(h1, 0.0)) - 1.0)
      h2 = jnp.dot(h1.astype(bf), w2_ref[...].astype(bf),
                   preferred_element_type=jnp.float32) + b2_ref[...]
      h2 = jnp.where(h2 > 0, h2, jnp.exp(jnp.minimum(h2, 0.0)) - 1.0)
      out_ref[...] = jnp.dot(h2.astype(bf), w3_ref[...].astype(bf),
                             preferred_element_type=jnp.float32) + b3_ref[...]

  return pl.pallas_call(
      body,
      grid=(NGRID_N,),
      in_specs=[
          pl.BlockSpec((TN, 64), lambda i: (i, 0)),
          pl.BlockSpec((TN, 1), lambda i: (i, 0)),
          pl.BlockSpec((64, 32), lambda i: (0, 0)),
          pl.BlockSpec((1, 32), lambda i: (0, 0)),
          pl.BlockSpec((32, 16), lambda i: (0, 0)),
          pl.BlockSpec((1, 16), lambda i: (0, 0)),
          pl.BlockSpec((16, 1), lambda i: (0, 0)),
          pl.BlockSpec((1, 1), lambda i: (0, 0)),
      ],
      out_specs=pl.BlockSpec((G, 1), lambda i: (0, 0)),
      out_shape=jax.ShapeDtypeStruct((G, 1), jnp.float32),
      scratch_shapes=[
          pltpu.VMEM((G, 64), jnp.float32),
          pltpu.VMEM((G, 1), jnp.float32),
      ],
  )(x3, batch2d, fc1_w, fc1_b, fc2_w, fc2_b, fc3_w, fc3_b)


def _prep_edge_mlp(w1, b1, w2, b2, m_in, dp, mo):
  w1p = jnp.pad(w1, ((0, 8 - w1.shape[0]), (0, 0)))
  b1r = b1.reshape(1, 128)
  w2p = jnp.pad(w2.reshape(128, m_in, mo), ((0, 0), (0, dp - m_in), (0, 0)))
  w2p = w2p.reshape(128, dp * mo)
  b2p = jnp.pad(b2.reshape(m_in, mo), ((0, dp - m_in), (0, 0)))
  b2p = b2p.reshape(1, dp * mo)
  return w1p, b1r, w2p, b2p


def kernel(x, edge_index, edge_attr, batch,
           nn1_w1, nn1_b1, nn1_w2, nn1_b2, root1, bias1,
           nn2_w1, nn2_b1, nn2_w2, nn2_b2, root2, bias2,
           nn3_w1, nn3_b1, nn3_w2, nn3_b2, root3, bias3,
           fc1_w, fc1_b, fc2_w, fc2_b, fc3_w, fc3_b):
  src = jnp.pad(edge_index[0], (0, E_PAD - E)).reshape(NW, NCH, CHUNK)
  dst = jnp.pad(edge_index[1], (0, E_PAD - E)).reshape(NW, NCH, CHUNK)
  eattr8 = jnp.pad(edge_attr, ((0, E_PAD - E), (0, 8 - edge_attr.shape[1])))
  batch2d = jnp.pad(batch, (0, N_PAD - N), constant_values=G).reshape(N_PAD, 1)

  layers = [
      (13, 16, 32, nn1_w1, nn1_b1, nn1_w2, nn1_b2, root1, bias1),
      (32, 32, 64, nn2_w1, nn2_b1, nn2_w2, nn2_b2, root2, bias2),
      (64, 64, 64, nn3_w1, nn3_b1, nn3_w2, nn3_b2, root3, bias3),
  ]

  xp = jnp.pad(x, ((0, N_PAD - N), (0, 3)))
  for m_in, dp, mo, w1, b1, w2, b2, root, bias in layers:
    w1p, b1r, w2p, b2p = _prep_edge_mlp(w1, b1, w2, b2, m_in, dp, mo)
    root_p = jnp.pad(root, ((0, dp - m_in), (0, 0)))
    xs = _sc_gather(xp, src, dp)
    msg = _tc_message(eattr8, xs, w1p, b1r, w2p, b2p, dp, mo)
    agg = _sc_scatter(msg, dst, jnp.zeros((N_PAD, mo), jnp.float32), mo)
    xp = _tc_update(xp, agg, root_p, bias.reshape(1, mo), dp, mo)

  out = _tc_pool_head(xp, batch2d,
                      fc1_w, fc1_b.reshape(1, 32),
                      fc2_w, fc2_b.reshape(1, 16),
                      fc3_w, fc3_b.reshape(1, 1))
  return out.reshape(-1)
